# baseline (speedup 1.0000x reference)
import functools

import jax
import jax.numpy as jnp
from jax import lax
from jax.experimental import pallas as pl
from jax.experimental.pallas import tpu as pltpu
from jax.experimental.pallas import tpu_sc as plsc

BATCH = 16384
N_FIELDS = 100
NUM_CORES = 2
NUM_SUBCORES = 16
NUM_WORKERS = NUM_CORES * NUM_SUBCORES
ROWS_PER_WORKER = BATCH // NUM_WORKERS
ROWS_PER_CHUNK = 128
NUM_CHUNKS = ROWS_PER_WORKER // ROWS_PER_CHUNK
CHUNK_ELEMS = ROWS_PER_CHUNK * N_FIELDS
LANES = 16
GROUPS = ROWS_PER_CHUNK // LANES

EMB_PAD = 1001472
TABLE_WORDS = 1000064
TABLE_SLICE = TABLE_WORDS // NUM_SUBCORES


def _sc_body(idx_hbm, val_hbm, emb_hbm, bias_hbm, out_hbm,
             idx_v, cid_v, val_v, gat_v0, gat_v1, out_v, bias_v,
             table_s, sem_i, sem_v, sem_g0, sem_g1, sem_t):
    sid = lax.axis_index("s")
    wid = sid * NUM_CORES + lax.axis_index("c")
    rbase = wid * ROWS_PER_WORKER

    pltpu.sync_copy(bias_hbm, bias_v)
    bias_vec = bias_v[...]

    gat_b = (gat_v0, gat_v1)
    sem_g = (sem_g0, sem_g1)

    def start_idx(c):
        return pltpu.async_copy(
            idx_hbm.at[:, pl.ds(rbase + c * ROWS_PER_CHUNK, ROWS_PER_CHUNK)],
            idx_v, sem_i)

    def compact_idx():
        def rbody(r, carry):
            for o in range(0, ROWS_PER_CHUNK, LANES):
                cid_v[pl.ds(r * ROWS_PER_CHUNK + o, LANES)] = (
                    idx_v[r, pl.ds(o, LANES)])
            return carry

        lax.fori_loop(0, N_FIELDS, rbody, 0)

    def start_val(c):
        return pltpu.async_copy(
            val_hbm.at[:, pl.ds(rbase + c * ROWS_PER_CHUNK, ROWS_PER_CHUNK)],
            val_v, sem_v)

    def start_gather(c):
        return pltpu.async_copy(table_s.at[cid_v], gat_b[c & 1],
                                sem_g[c & 1])

    hi = {0: start_idx(0)}
    hv = {0: start_val(0)}
    hg = {}

    tbase = sid * TABLE_SLICE
    pieces = [(k * CHUNK_ELEMS, CHUNK_ELEMS)
              for k in range(TABLE_SLICE // CHUNK_ELEMS)]
    tail = TABLE_SLICE % CHUNK_ELEMS
    if tail:
        pieces.append((TABLE_SLICE - tail, tail))
    hs = pltpu.async_copy(emb_hbm.at[pl.ds(tbase, pieces[0][1])],
                          gat_v0.at[pl.ds(0, pieces[0][1])], sem_t)
    for k, (off, sz) in enumerate(pieces):
        buf = gat_b[k & 1]
        hs.wait()
        if k + 1 < len(pieces):
            noff, nsz = pieces[k + 1]
            hs = pltpu.async_copy(emb_hbm.at[pl.ds(tbase + noff, nsz)],
                                  gat_b[(k + 1) & 1].at[pl.ds(0, nsz)],
                                  sem_t)
        pltpu.sync_copy(buf.at[pl.ds(0, sz)],
                        table_s.at[pl.ds(tbase + off, sz)])
    hi[0].wait()
    compact_idx()
    hi[1] = start_idx(1)
    plsc.subcore_barrier()
    hg[0] = start_gather(0)

    for c in range(NUM_CHUNKS):
        cur = c & 1
        hg[c].wait()
        if c + 1 < NUM_CHUNKS:
            hi[c + 1].wait()
            compact_idx()
            hg[c + 1] = start_gather(c + 1)
            if c + 2 < NUM_CHUNKS:
                hi[c + 2] = start_idx(c + 2)
        hv[c].wait()
        gat, val = gat_b[cur], val_v
        for grp in range(GROUPS):
            col = grp * LANES

            def fbody(f, acc, _col=col, _gat=gat, _val=val):
                g = gat[pl.ds(f * ROWS_PER_CHUNK + _col, LANES)]
                v = _val[f, pl.ds(_col, LANES)]
                return acc + g * v

            acc = lax.fori_loop(0, N_FIELDS, fbody,
                                jnp.zeros((LANES,), jnp.float32))
            out_v[pl.ds(c * ROWS_PER_CHUNK + col, LANES)] = acc + bias_vec
        if c + 1 < NUM_CHUNKS:
            hv[c + 1] = start_val(c + 1)

    pltpu.sync_copy(out_v, out_hbm.at[pl.ds(rbase, ROWS_PER_WORKER)])


_sc_call = functools.partial(
    pl.kernel,
    out_type=jax.ShapeDtypeStruct((BATCH,), jnp.float32),
    mesh=plsc.VectorSubcoreMesh(core_axis_name="c", subcore_axis_name="s"),
    compiler_params=pltpu.CompilerParams(needs_layout_passes=False),
    scratch_types=[
        pltpu.VMEM((N_FIELDS, ROWS_PER_CHUNK), jnp.int32),
        pltpu.VMEM((CHUNK_ELEMS,), jnp.int32),
        pltpu.VMEM((N_FIELDS, ROWS_PER_CHUNK), jnp.float32),
        pltpu.VMEM((CHUNK_ELEMS,), jnp.float32),
        pltpu.VMEM((CHUNK_ELEMS,), jnp.float32),
        pltpu.VMEM((ROWS_PER_WORKER,), jnp.float32),
        pltpu.VMEM((LANES,), jnp.float32),
        pltpu.VMEM_SHARED((TABLE_WORDS,), jnp.float32),
        pltpu.SemaphoreType.DMA,
        pltpu.SemaphoreType.DMA,
        pltpu.SemaphoreType.DMA,
        pltpu.SemaphoreType.DMA,
        pltpu.SemaphoreType.DMA,
    ],
)(_sc_body)


def kernel(index, value, emb, bias):
    idx_t = index.T
    val_t = value.T
    emb1 = jnp.pad(emb.T, ((0, 0), (0, EMB_PAD - emb.shape[0]))).reshape(-1)
    bias16 = jnp.broadcast_to(bias, (LANES,))
    return _sc_call(idx_t, val_t, emb1, bias16)


# speedup vs baseline: 1.0133x; 1.0133x over previous
import functools

import jax
import jax.numpy as jnp
from jax import lax
from jax.experimental import pallas as pl
from jax.experimental.pallas import tpu as pltpu
from jax.experimental.pallas import tpu_sc as plsc

BATCH = 16384
N_FIELDS = 100
NUM_CORES = 2
NUM_SUBCORES = 16
NUM_WORKERS = NUM_CORES * NUM_SUBCORES
ROWS_PER_WORKER = BATCH // NUM_WORKERS
ROWS_PER_CHUNK = 128
NUM_CHUNKS = ROWS_PER_WORKER // ROWS_PER_CHUNK
CHUNK_ELEMS = ROWS_PER_CHUNK * N_FIELDS
LANES = 16
GROUPS = ROWS_PER_CHUNK // LANES

EMB_PAD = 1001472
TABLE_WORDS = 1000064
TABLE_SLICE = TABLE_WORDS // NUM_SUBCORES


def _sc_body(idx_hbm, val_hbm, emb_hbm, bias_hbm, out_hbm,
             idx_v, cid_v, val_v, gat_v0, gat_v1, out_v, bias_v,
             table_s, sem_i, sem_v, sem_g0, sem_g1, sem_t):
    sid = lax.axis_index("s")
    wid = sid * NUM_CORES + lax.axis_index("c")
    rbase = wid * ROWS_PER_WORKER

    pltpu.sync_copy(bias_hbm, bias_v)
    bias_vec = bias_v[...]

    gat_b = (gat_v0, gat_v1)
    sem_g = (sem_g0, sem_g1)

    def start_idx(c):
        return pltpu.async_copy(
            idx_hbm.at[:, pl.ds(rbase + c * ROWS_PER_CHUNK, ROWS_PER_CHUNK)],
            idx_v, sem_i)

    def compact_idx():
        def rbody(r, carry):
            for o in range(0, ROWS_PER_CHUNK, LANES):
                cid_v[pl.ds(r * ROWS_PER_CHUNK + o, LANES)] = (
                    idx_v[r, pl.ds(o, LANES)])
            return carry

        lax.fori_loop(0, N_FIELDS, rbody, 0)

    def start_val(c):
        return pltpu.async_copy(
            val_hbm.at[:, pl.ds(rbase + c * ROWS_PER_CHUNK, ROWS_PER_CHUNK)],
            val_v, sem_v)

    def start_gather(c):
        return pltpu.async_copy(table_s.at[cid_v], gat_b[c & 1],
                                sem_g[c & 1])

    hg = {}

    tbase = sid * TABLE_SLICE
    pieces = [(k * CHUNK_ELEMS, CHUNK_ELEMS)
              for k in range(TABLE_SLICE // CHUNK_ELEMS)]
    tail = TABLE_SLICE % CHUNK_ELEMS
    if tail:
        pieces.append((TABLE_SLICE - tail, tail))
    hs = pltpu.async_copy(emb_hbm.at[pl.ds(tbase, pieces[0][1])],
                          gat_v0.at[pl.ds(0, pieces[0][1])], sem_t)
    hi = {0: start_idx(0)}
    hv = {}
    for k, (off, sz) in enumerate(pieces):
        buf = gat_b[k & 1]
        hs.wait()
        if k + 1 < len(pieces):
            noff, nsz = pieces[k + 1]
            hs = pltpu.async_copy(emb_hbm.at[pl.ds(tbase + noff, nsz)],
                                  gat_b[(k + 1) & 1].at[pl.ds(0, nsz)],
                                  sem_t)
        pltpu.sync_copy(buf.at[pl.ds(0, sz)],
                        table_s.at[pl.ds(tbase + off, sz)])
    hv[0] = start_val(0)
    hi[0].wait()
    compact_idx()
    hi[1] = start_idx(1)
    plsc.subcore_barrier()
    hg[0] = start_gather(0)

    for c in range(NUM_CHUNKS):
        cur = c & 1
        hg[c].wait()
        if c + 1 < NUM_CHUNKS:
            hi[c + 1].wait()
            compact_idx()
            hg[c + 1] = start_gather(c + 1)
            if c + 2 < NUM_CHUNKS:
                hi[c + 2] = start_idx(c + 2)
        hv[c].wait()
        gat, val = gat_b[cur], val_v
        for grp in range(GROUPS):
            col = grp * LANES

            def fbody(f, acc, _col=col, _gat=gat, _val=val):
                g = gat[pl.ds(f * ROWS_PER_CHUNK + _col, LANES)]
                v = _val[f, pl.ds(_col, LANES)]
                return acc + g * v

            acc = lax.fori_loop(0, N_FIELDS, fbody,
                                jnp.zeros((LANES,), jnp.float32))
            out_v[pl.ds(c * ROWS_PER_CHUNK + col, LANES)] = acc + bias_vec
        if c + 1 < NUM_CHUNKS:
            hv[c + 1] = start_val(c + 1)

    pltpu.sync_copy(out_v, out_hbm.at[pl.ds(rbase, ROWS_PER_WORKER)])


_sc_call = functools.partial(
    pl.kernel,
    out_type=jax.ShapeDtypeStruct((BATCH,), jnp.float32),
    mesh=plsc.VectorSubcoreMesh(core_axis_name="c", subcore_axis_name="s"),
    compiler_params=pltpu.CompilerParams(needs_layout_passes=False),
    scratch_types=[
        pltpu.VMEM((N_FIELDS, ROWS_PER_CHUNK), jnp.int32),
        pltpu.VMEM((CHUNK_ELEMS,), jnp.int32),
        pltpu.VMEM((N_FIELDS, ROWS_PER_CHUNK), jnp.float32),
        pltpu.VMEM((CHUNK_ELEMS,), jnp.float32),
        pltpu.VMEM((CHUNK_ELEMS,), jnp.float32),
        pltpu.VMEM((ROWS_PER_WORKER,), jnp.float32),
        pltpu.VMEM((LANES,), jnp.float32),
        pltpu.VMEM_SHARED((TABLE_WORDS,), jnp.float32),
        pltpu.SemaphoreType.DMA,
        pltpu.SemaphoreType.DMA,
        pltpu.SemaphoreType.DMA,
        pltpu.SemaphoreType.DMA,
        pltpu.SemaphoreType.DMA,
    ],
)(_sc_body)


def kernel(index, value, emb, bias):
    idx_t = index.T
    val_t = value.T
    emb1 = jnp.pad(emb.T, ((0, 0), (0, EMB_PAD - emb.shape[0]))).reshape(-1)
    bias16 = jnp.broadcast_to(bias, (LANES,))
    return _sc_call(idx_t, val_t, emb1, bias16)
